# baseline (speedup 1.0000x reference)
import functools
import math

import jax
import jax.numpy as jnp
from jax import lax
from jax.experimental import pallas as pl
from jax.experimental.pallas import tpu as pltpu
from jax.experimental.pallas import tpu_sc as plsc


def _build_c_matrices(in_src, in_dst, bias_in, con_src, con_dst, bias_con, M, N):
    info = plsc.get_sparse_core_info()
    NC, NS = info.num_cores, info.num_subcores
    F1 = M * N
    F2 = N * M
    CH = 1_280_000
    PT = CH // NS
    ZB = 8_000
    ZREP = PT // ZB
    NCH1 = -(-F1 // CH)
    NCH2 = -(-F2 // CH)

    E = in_src.shape[0]
    lanes = 16
    epad = (-E) % (NS * lanes)
    if epad:
        pad_i = jnp.zeros((epad,), in_src.dtype)
        pad_f = jnp.full((epad,), -1e30, jnp.float32)
        in_src = jnp.concatenate([in_src, pad_i])
        in_dst = jnp.concatenate([in_dst, pad_i])
        con_src = jnp.concatenate([con_src, pad_i])
        con_dst = jnp.concatenate([con_dst, pad_i])
        bias_in = jnp.concatenate([bias_in, pad_f])
        bias_con = jnp.concatenate([bias_con, pad_f])
    EP = (E + epad) // NS
    LOOPS = EP // lanes
    IDXP = ((EP + 127) // 128) * 128

    def body(insrc_h, indst_h, bin_h, consrc_h, condst_h, bcon_h,
             c1_h, c2_h, flat, idx, val, zeros, stage, shared):
        c = lax.axis_index("c")
        s = lax.axis_index("s")
        z16 = jnp.zeros((16,), jnp.float32)
        dvec = CH + 16 * s + lax.iota(jnp.int32, 16)

        def zfill(i, carry):
            zeros[pl.ds(16 * i, 16)] = z16
            return carry
        lax.fori_loop(0, ZB // 16, zfill, 0)
        for t in range(IDXP // 16 - LOOPS):
            val[pl.ds(EP + 16 * t, 16)] = z16
            idx[pl.ds(EP + 16 * t, 16)] = dvec

        for mat in range(2):
            src_h = insrc_h if mat == 0 else consrc_h
            dst_h = indst_h if mat == 0 else condst_h
            b_h = bin_h if mat == 0 else bcon_h
            out_h = c1_h if mat == 0 else c2_h
            ncols = N if mat == 0 else M
            nch = NCH1 if mat == 0 else NCH2
            per_core = -(-nch // NC)

            pltpu.sync_copy(dst_h.at[pl.ds(s * EP, EP)], flat)
            pltpu.sync_copy(src_h.at[pl.ds(s * EP, EP)], idx.at[pl.ds(0, EP)])
            pltpu.sync_copy(b_h.at[pl.ds(s * EP, EP)], val.at[pl.ds(0, EP)])

            def build(i, carry):
                o = 16 * i
                flat[pl.ds(o, 16)] = flat[pl.ds(o, 16)] * ncols + idx[pl.ds(o, 16)]
                val[pl.ds(o, 16)] = jnp.exp(val[pl.ds(o, 16)])
                return carry
            lax.fori_loop(0, LOOPS, build, 0)

            for jj in range(per_core):
                j = jnp.minimum(per_core * c + jj, nch - 1)
                lo = j * CH
                for z in range(ZREP):
                    pltpu.sync_copy(zeros.at[pl.ds(0, ZB)],
                                    shared.at[pl.ds(s * PT + z * ZB, ZB)])
                plsc.subcore_barrier()

                def mkidx(i, carry):
                    o = 16 * i
                    f16 = flat[pl.ds(o, 16)]
                    inr = (f16 >= lo) & (f16 < lo + CH)
                    idx[pl.ds(o, 16)] = jnp.where(inr, f16 - lo, dvec)
                    return carry
                lax.fori_loop(0, LOOPS, mkidx, 0)

                pltpu.sync_copy(val, shared.at[idx], add=True)
                plsc.subcore_barrier()
                for z in range(ZREP):
                    off = s * PT + z * ZB
                    pltpu.sync_copy(shared.at[pl.ds(off, ZB)], stage)
                    pltpu.sync_copy(stage, out_h.at[pl.ds(lo + off, ZB)])

    builder = pl.kernel(
        body,
        out_type=[jax.ShapeDtypeStruct((NCH1 * CH,), jnp.float32),
                  jax.ShapeDtypeStruct((NCH2 * CH,), jnp.float32)],
        mesh=plsc.VectorSubcoreMesh(core_axis_name="c", subcore_axis_name="s"),
        scratch_types=[
            pltpu.VMEM((EP,), jnp.int32),
            pltpu.VMEM((IDXP,), jnp.int32),
            pltpu.VMEM((IDXP,), jnp.float32),
            pltpu.VMEM((ZB,), jnp.float32),
            pltpu.VMEM((ZB,), jnp.float32),
            pltpu.VMEM_SHARED((CH + 384,), jnp.float32),
        ],
    )
    c1f, c2f = builder(in_src, in_dst, bias_in, con_src, con_dst, bias_con)
    return c1f[:F1].reshape(M, N), c2f[:F2].reshape(N, M)


def _ln(x, g, b):
    m = jnp.mean(x, axis=-1, keepdims=True)
    v = jnp.mean((x - m) * (x - m), axis=-1, keepdims=True)
    return (x - m) * jax.lax.rsqrt(v + 1e-5) * g + b


def _vtx_body(K, vfeat_ref, cent_ref, uniq_ref, Wv_ref, bv_ref, cs_ref, un_ref,
              Wk_ref, bk_ref, Wvv_ref, bvv_ref, Wq2_ref, bq2_ref,
              feat_ref, k_ref, v_ref, q2_ref):
    x = vfeat_ref[...]
    R = x.shape[0]
    f = jnp.dot(x, Wv_ref[...], preferred_element_type=jnp.float32) + bv_ref[...]
    oh_c = (jax.lax.broadcasted_iota(jnp.int32, (R, K), 1) == cent_ref[...]).astype(jnp.float32)
    f = f + jnp.dot(oh_c, cs_ref[...], preferred_element_type=jnp.float32)
    oh_u = (jax.lax.broadcasted_iota(jnp.int32, (R, K), 1) == uniq_ref[...]).astype(jnp.float32)
    f = f + jnp.dot(oh_u, un_ref[...], preferred_element_type=jnp.float32)
    feat_ref[...] = f
    k_ref[...] = jnp.dot(f, Wk_ref[...], preferred_element_type=jnp.float32) + bk_ref[...]
    v_ref[...] = jnp.dot(f, Wvv_ref[...], preferred_element_type=jnp.float32) + bvv_ref[...]
    q2_ref[...] = jnp.dot(f, Wq2_ref[...], preferred_element_type=jnp.float32) + bq2_ref[...]


def _edge_body(inv_sqrt_qd,
               efeat_ref, k_ref, v_ref, C_ref,
               Wqe_ref, bqe_ref, Wl1_ref, bl1_ref, Wl2_ref, bl2_ref,
               ln1g_ref, ln1b_ref, Wke_ref, bke_ref, Wve_ref, bve_ref,
               feat_e_ref, k2_ref, v2_ref):
    ef = efeat_ref[...]
    q = jnp.dot(ef, Wqe_ref[...], preferred_element_type=jnp.float32) + bqe_ref[...]
    S = jax.lax.dot_general(q, k_ref[...], (((1,), (1,)), ((), ())),
                            preferred_element_type=jnp.float32)
    A = jnp.where(S >= 0, S, 0.01 * S) * inv_sqrt_qd
    m = jnp.max(A, axis=1, keepdims=True)
    P = jnp.exp(A - m) * C_ref[...]
    s = jnp.sum(P, axis=1, keepdims=True)
    h = jnp.dot(P, v_ref[...], preferred_element_type=jnp.float32) / jnp.maximum(s, 1e-30)
    x = _ln(h + ef, ln1g_ref[...], ln1b_ref[...])
    f = jnp.dot(jax.nn.relu(jnp.dot(x, Wl1_ref[...], preferred_element_type=jnp.float32) + bl1_ref[...]),
                Wl2_ref[...], preferred_element_type=jnp.float32) + bl2_ref[...]
    fe = _ln(f + x, ln1g_ref[...], ln1b_ref[...])
    feat_e_ref[...] = fe
    k2_ref[...] = jnp.dot(fe, Wke_ref[...], preferred_element_type=jnp.float32) + bke_ref[...]
    v2_ref[...] = jnp.dot(fe, Wve_ref[...], preferred_element_type=jnp.float32) + bve_ref[...]


def _node_body(inv_sqrt_qd,
               featv_ref, q2_ref, k2_ref, v2_ref, C_ref,
               Wl3_ref, bl3_ref, Wl4_ref, bl4_ref,
               ln2g_ref, ln2b_ref, Wmlp_ref, bmlp_ref,
               out_ref):
    S = jax.lax.dot_general(q2_ref[...], k2_ref[...], (((1,), (1,)), ((), ())),
                            preferred_element_type=jnp.float32)
    A = jnp.where(S >= 0, S, 0.01 * S) * inv_sqrt_qd
    m = jnp.max(A, axis=1, keepdims=True)
    P = jnp.exp(A - m) * C_ref[...]
    s = jnp.sum(P, axis=1, keepdims=True)
    h = jnp.dot(P, v2_ref[...], preferred_element_type=jnp.float32) / jnp.maximum(s, 1e-30)
    y = _ln(h + featv_ref[...], ln2g_ref[...], ln2b_ref[...])
    f2 = jnp.dot(jax.nn.relu(jnp.dot(y, Wl3_ref[...], preferred_element_type=jnp.float32) + bl3_ref[...]),
                 Wl4_ref[...], preferred_element_type=jnp.float32) + bl4_ref[...]
    fv2 = _ln(f2 + y, ln2g_ref[...], ln2b_ref[...])
    out_ref[...] = jnp.dot(fv2, Wmlp_ref[...], preferred_element_type=jnp.float32) + bmlp_ref[...]


def _full(shape):
    return pl.BlockSpec(shape, lambda i: (0,) * len(shape))


def kernel(vfeat, efeat, bias_in, bias_con, W_vtx1, b_vtx1, cs_emb, un_emb,
           W_kv, b_kv, W_vv, b_vv, W_qe, b_qe, W_ke, b_ke, W_ve, b_ve,
           W_qv, b_qv, ln1_g, ln1_b, ln2_g, ln2_b, W_l1, b_l1, W_l2, b_l2,
           W_l3, b_l3, W_l4, b_l4, W_mlp, b_mlp,
           centrality_values, uniqueness, in_src, in_dst, con_src, con_dst):
    N, D = vfeat.shape
    M = efeat.shape[0]
    K = cs_emb.shape[0]
    H = W_mlp.shape[1]
    qd = W_kv.shape[1]
    inv_sqrt_qd = 1.0 / math.sqrt(qd)

    r2 = lambda a: a.reshape(1, -1)

    C1, C2 = _build_c_matrices(in_src, in_dst, bias_in,
                               con_src, con_dst, bias_con, M, N)

    NB = 1000 if N % 1000 == 0 else N
    grid_n = N // NB
    cent2 = centrality_values.reshape(N, 1)
    uniq2 = uniqueness.reshape(N, 1)
    row_spec = pl.BlockSpec((NB, D), lambda i: (i, 0))
    idx_spec = pl.BlockSpec((NB, 1), lambda i: (i, 0))
    feat_v, kv, vv, q2 = pl.pallas_call(
        functools.partial(_vtx_body, K),
        grid=(grid_n,),
        in_specs=[row_spec, idx_spec, idx_spec,
                  _full((D, D)), _full((1, D)), _full((K, D)), _full((K, D)),
                  _full((D, D)), _full((1, D)), _full((D, D)), _full((1, D)),
                  _full((D, D)), _full((1, D))],
        out_specs=[row_spec, row_spec, row_spec, row_spec],
        out_shape=[jax.ShapeDtypeStruct((N, D), jnp.float32)] * 4,
    )(vfeat, cent2, uniq2, W_vtx1, r2(b_vtx1), cs_emb, un_emb,
      W_kv, r2(b_kv), W_vv, r2(b_vv), W_qv, r2(b_qv))

    MB = 80 if M % 80 == 0 else M
    grid_m = M // MB
    mrow_spec = pl.BlockSpec((MB, D), lambda i: (i, 0))
    feat_e, k2, v2 = pl.pallas_call(
        functools.partial(_edge_body, inv_sqrt_qd),
        grid=(grid_m,),
        in_specs=[mrow_spec, _full((N, D)), _full((N, D)),
                  pl.BlockSpec((MB, N), lambda i: (i, 0)),
                  _full((D, D)), _full((1, D)), _full((D, D)), _full((1, D)),
                  _full((D, D)), _full((1, D)), _full((1, D)), _full((1, D)),
                  _full((D, D)), _full((1, D)), _full((D, D)), _full((1, D))],
        out_specs=[mrow_spec, mrow_spec, mrow_spec],
        out_shape=[jax.ShapeDtypeStruct((M, D), jnp.float32)] * 3,
    )(efeat, kv, vv, C1,
      W_qe, r2(b_qe), W_l1, r2(b_l1), W_l2, r2(b_l2), r2(ln1_g), r2(ln1_b),
      W_ke, r2(b_ke), W_ve, r2(b_ve))

    out = pl.pallas_call(
        functools.partial(_node_body, inv_sqrt_qd),
        grid=(grid_n,),
        in_specs=[row_spec, row_spec, _full((M, D)), _full((M, D)),
                  pl.BlockSpec((NB, M), lambda i: (i, 0)),
                  _full((D, D)), _full((1, D)), _full((D, D)), _full((1, D)),
                  _full((1, D)), _full((1, D)), _full((D, H)), _full((1, H))],
        out_specs=pl.BlockSpec((NB, H), lambda i: (i, 0)),
        out_shape=jax.ShapeDtypeStruct((N, H), jnp.float32),
    )(feat_v, q2, k2, v2, C2,
      W_l3, r2(b_l3), W_l4, r2(b_l4), r2(ln2_g), r2(ln2_b), W_mlp, r2(b_mlp))

    return out


# speedup vs baseline: 1.1391x; 1.1391x over previous
import functools
import math

import jax
import jax.numpy as jnp
from jax import lax
from jax.experimental import pallas as pl
from jax.experimental.pallas import tpu as pltpu
from jax.experimental.pallas import tpu_sc as plsc


def _build_c_matrices(in_src, in_dst, bias_in, con_src, con_dst, bias_con, M, N):
    info = plsc.get_sparse_core_info()
    NC, NS = info.num_cores, info.num_subcores
    F1 = M * N
    F2 = N * M
    CH = 1_024_000
    PT = CH // NS
    ZB = 16_000
    ZREP = PT // ZB
    NCH1 = -(-F1 // CH)
    NCH2 = -(-F2 // CH)
    exact1 = F1 % ZB == 0
    exact2 = F2 % ZB == 0
    O1 = F1 if exact1 else NCH1 * CH
    O2 = F2 if exact2 else NCH2 * CH

    E = in_src.shape[0]
    lanes = 16
    epad = (-E) % (NS * lanes)
    if epad:
        pad_i = jnp.zeros((epad,), in_src.dtype)
        pad_f = jnp.full((epad,), -1e30, jnp.float32)
        in_src = jnp.concatenate([in_src, pad_i])
        in_dst = jnp.concatenate([in_dst, pad_i])
        con_src = jnp.concatenate([con_src, pad_i])
        con_dst = jnp.concatenate([con_dst, pad_i])
        bias_in = jnp.concatenate([bias_in, pad_f])
        bias_con = jnp.concatenate([bias_con, pad_f])
    EP = (E + epad) // NS
    LOOPS = EP // lanes
    IDXP = ((EP + 127) // 128) * 128

    def body(insrc_h, indst_h, bin_h, consrc_h, condst_h, bcon_h,
             c1_h, c2_h, flat, idx, val, zeros, stage, shared):
        c = lax.axis_index("c")
        s = lax.axis_index("s")
        z16 = jnp.zeros((16,), jnp.float32)
        dvec = CH + 16 * s + lax.iota(jnp.int32, 16)

        def zfill(i, carry):
            zeros[pl.ds(16 * i, 16)] = z16
            return carry
        lax.fori_loop(0, ZB // 16, zfill, 0)
        for t in range(IDXP // 16 - LOOPS):
            val[pl.ds(EP + 16 * t, 16)] = z16
            idx[pl.ds(EP + 16 * t, 16)] = dvec

        for mat in range(2):
            src_h = insrc_h if mat == 0 else consrc_h
            dst_h = indst_h if mat == 0 else condst_h
            b_h = bin_h if mat == 0 else bcon_h
            out_h = c1_h if mat == 0 else c2_h
            ncols = N if mat == 0 else M
            nch = NCH1 if mat == 0 else NCH2
            fsz = O1 if mat == 0 else O2
            per_core = -(-nch // NC)

            pltpu.sync_copy(dst_h.at[pl.ds(s * EP, EP)], flat)
            pltpu.sync_copy(src_h.at[pl.ds(s * EP, EP)], idx.at[pl.ds(0, EP)])
            pltpu.sync_copy(b_h.at[pl.ds(s * EP, EP)], val.at[pl.ds(0, EP)])

            def build(i, carry):
                o = 16 * i
                flat[pl.ds(o, 16)] = flat[pl.ds(o, 16)] * ncols + idx[pl.ds(o, 16)]
                val[pl.ds(o, 16)] = jnp.exp(val[pl.ds(o, 16)])
                return carry
            lax.fori_loop(0, LOOPS, build, 0)

            for jj in range(per_core):
                j = jnp.minimum(per_core * c + jj, nch - 1)
                lo = j * CH
                for z in range(ZREP):
                    pltpu.sync_copy(zeros.at[pl.ds(0, ZB)],
                                    shared.at[pl.ds(s * PT + z * ZB, ZB)])
                plsc.subcore_barrier()

                def mkidx(i, carry):
                    o = 16 * i
                    f16 = flat[pl.ds(o, 16)]
                    inr = (f16 >= lo) & (f16 < lo + CH)
                    idx[pl.ds(o, 16)] = jnp.where(inr, f16 - lo, dvec)
                    return carry
                lax.fori_loop(0, LOOPS, mkidx, 0)

                pltpu.sync_copy(val, shared.at[idx], add=True)
                plsc.subcore_barrier()
                for z in range(ZREP):
                    off = s * PT + z * ZB

                    @pl.when(lo + off + ZB <= fsz)
                    def _():
                        pltpu.sync_copy(shared.at[pl.ds(off, ZB)], stage)
                        pltpu.sync_copy(stage, out_h.at[pl.ds(lo + off, ZB)])

    builder = pl.kernel(
        body,
        out_type=[jax.ShapeDtypeStruct((O1,), jnp.float32),
                  jax.ShapeDtypeStruct((O2,), jnp.float32)],
        mesh=plsc.VectorSubcoreMesh(core_axis_name="c", subcore_axis_name="s"),
        scratch_types=[
            pltpu.VMEM((EP,), jnp.int32),
            pltpu.VMEM((IDXP,), jnp.int32),
            pltpu.VMEM((IDXP,), jnp.float32),
            pltpu.VMEM((ZB,), jnp.float32),
            pltpu.VMEM((ZB,), jnp.float32),
            pltpu.VMEM_SHARED((CH + 384,), jnp.float32),
        ],
    )
    c1f, c2f = builder(in_src, in_dst, bias_in, con_src, con_dst, bias_con)
    return c1f[:F1].reshape(M, N), c2f[:F2].reshape(N, M)


def _ln(x, g, b):
    m = jnp.mean(x, axis=-1, keepdims=True)
    v = jnp.mean((x - m) * (x - m), axis=-1, keepdims=True)
    return (x - m) * jax.lax.rsqrt(v + 1e-5) * g + b


def _vtx_body(K, vfeat_ref, cent_ref, uniq_ref, Wv_ref, bv_ref, cs_ref, un_ref,
              Wk_ref, bk_ref, Wvv_ref, bvv_ref, Wq2_ref, bq2_ref,
              feat_ref, k_ref, v_ref, q2_ref):
    x = vfeat_ref[...]
    R = x.shape[0]
    f = jnp.dot(x, Wv_ref[...], preferred_element_type=jnp.float32) + bv_ref[...]
    oh_c = (jax.lax.broadcasted_iota(jnp.int32, (R, K), 1) == cent_ref[...]).astype(jnp.float32)
    f = f + jnp.dot(oh_c, cs_ref[...], preferred_element_type=jnp.float32)
    oh_u = (jax.lax.broadcasted_iota(jnp.int32, (R, K), 1) == uniq_ref[...]).astype(jnp.float32)
    f = f + jnp.dot(oh_u, un_ref[...], preferred_element_type=jnp.float32)
    feat_ref[...] = f
    k_ref[...] = jnp.dot(f, Wk_ref[...], preferred_element_type=jnp.float32) + bk_ref[...]
    v_ref[...] = jnp.dot(f, Wvv_ref[...], preferred_element_type=jnp.float32) + bvv_ref[...]
    q2_ref[...] = jnp.dot(f, Wq2_ref[...], preferred_element_type=jnp.float32) + bq2_ref[...]


def _edge_body(inv_sqrt_qd,
               efeat_ref, k_ref, v_ref, C_ref,
               Wqe_ref, bqe_ref, Wl1_ref, bl1_ref, Wl2_ref, bl2_ref,
               ln1g_ref, ln1b_ref, Wke_ref, bke_ref, Wve_ref, bve_ref,
               feat_e_ref, k2_ref, v2_ref):
    ef = efeat_ref[...]
    q = jnp.dot(ef, Wqe_ref[...], preferred_element_type=jnp.float32) + bqe_ref[...]
    S = jax.lax.dot_general(q, k_ref[...], (((1,), (1,)), ((), ())),
                            preferred_element_type=jnp.float32)
    A = jnp.where(S >= 0, S, 0.01 * S) * inv_sqrt_qd
    m = jnp.max(A, axis=1, keepdims=True)
    P = jnp.exp(A - m) * C_ref[...]
    s = jnp.sum(P, axis=1, keepdims=True)
    h = jnp.dot(P, v_ref[...], preferred_element_type=jnp.float32) / jnp.maximum(s, 1e-30)
    x = _ln(h + ef, ln1g_ref[...], ln1b_ref[...])
    f = jnp.dot(jax.nn.relu(jnp.dot(x, Wl1_ref[...], preferred_element_type=jnp.float32) + bl1_ref[...]),
                Wl2_ref[...], preferred_element_type=jnp.float32) + bl2_ref[...]
    fe = _ln(f + x, ln1g_ref[...], ln1b_ref[...])
    feat_e_ref[...] = fe
    k2_ref[...] = jnp.dot(fe, Wke_ref[...], preferred_element_type=jnp.float32) + bke_ref[...]
    v2_ref[...] = jnp.dot(fe, Wve_ref[...], preferred_element_type=jnp.float32) + bve_ref[...]


def _node_body(inv_sqrt_qd,
               featv_ref, q2_ref, k2_ref, v2_ref, C_ref,
               Wl3_ref, bl3_ref, Wl4_ref, bl4_ref,
               ln2g_ref, ln2b_ref, Wmlp_ref, bmlp_ref,
               out_ref):
    S = jax.lax.dot_general(q2_ref[...], k2_ref[...], (((1,), (1,)), ((), ())),
                            preferred_element_type=jnp.float32)
    A = jnp.where(S >= 0, S, 0.01 * S) * inv_sqrt_qd
    m = jnp.max(A, axis=1, keepdims=True)
    P = jnp.exp(A - m) * C_ref[...]
    s = jnp.sum(P, axis=1, keepdims=True)
    h = jnp.dot(P, v2_ref[...], preferred_element_type=jnp.float32) / jnp.maximum(s, 1e-30)
    y = _ln(h + featv_ref[...], ln2g_ref[...], ln2b_ref[...])
    f2 = jnp.dot(jax.nn.relu(jnp.dot(y, Wl3_ref[...], preferred_element_type=jnp.float32) + bl3_ref[...]),
                 Wl4_ref[...], preferred_element_type=jnp.float32) + bl4_ref[...]
    fv2 = _ln(f2 + y, ln2g_ref[...], ln2b_ref[...])
    out_ref[...] = jnp.dot(fv2, Wmlp_ref[...], preferred_element_type=jnp.float32) + bmlp_ref[...]


def _full(shape):
    return pl.BlockSpec(shape, lambda i: (0,) * len(shape))


def kernel(vfeat, efeat, bias_in, bias_con, W_vtx1, b_vtx1, cs_emb, un_emb,
           W_kv, b_kv, W_vv, b_vv, W_qe, b_qe, W_ke, b_ke, W_ve, b_ve,
           W_qv, b_qv, ln1_g, ln1_b, ln2_g, ln2_b, W_l1, b_l1, W_l2, b_l2,
           W_l3, b_l3, W_l4, b_l4, W_mlp, b_mlp,
           centrality_values, uniqueness, in_src, in_dst, con_src, con_dst):
    N, D = vfeat.shape
    M = efeat.shape[0]
    K = cs_emb.shape[0]
    H = W_mlp.shape[1]
    qd = W_kv.shape[1]
    inv_sqrt_qd = 1.0 / math.sqrt(qd)

    r2 = lambda a: a.reshape(1, -1)

    C1, C2 = _build_c_matrices(in_src, in_dst, bias_in,
                               con_src, con_dst, bias_con, M, N)

    NB = 1000 if N % 1000 == 0 else N
    grid_n = N // NB
    cent2 = centrality_values.reshape(N, 1)
    uniq2 = uniqueness.reshape(N, 1)
    row_spec = pl.BlockSpec((NB, D), lambda i: (i, 0))
    idx_spec = pl.BlockSpec((NB, 1), lambda i: (i, 0))
    feat_v, kv, vv, q2 = pl.pallas_call(
        functools.partial(_vtx_body, K),
        grid=(grid_n,),
        in_specs=[row_spec, idx_spec, idx_spec,
                  _full((D, D)), _full((1, D)), _full((K, D)), _full((K, D)),
                  _full((D, D)), _full((1, D)), _full((D, D)), _full((1, D)),
                  _full((D, D)), _full((1, D))],
        out_specs=[row_spec, row_spec, row_spec, row_spec],
        out_shape=[jax.ShapeDtypeStruct((N, D), jnp.float32)] * 4,
    )(vfeat, cent2, uniq2, W_vtx1, r2(b_vtx1), cs_emb, un_emb,
      W_kv, r2(b_kv), W_vv, r2(b_vv), W_qv, r2(b_qv))

    MB = 80 if M % 80 == 0 else M
    grid_m = M // MB
    mrow_spec = pl.BlockSpec((MB, D), lambda i: (i, 0))
    feat_e, k2, v2 = pl.pallas_call(
        functools.partial(_edge_body, inv_sqrt_qd),
        grid=(grid_m,),
        in_specs=[mrow_spec, _full((N, D)), _full((N, D)),
                  pl.BlockSpec((MB, N), lambda i: (i, 0)),
                  _full((D, D)), _full((1, D)), _full((D, D)), _full((1, D)),
                  _full((D, D)), _full((1, D)), _full((1, D)), _full((1, D)),
                  _full((D, D)), _full((1, D)), _full((D, D)), _full((1, D))],
        out_specs=[mrow_spec, mrow_spec, mrow_spec],
        out_shape=[jax.ShapeDtypeStruct((M, D), jnp.float32)] * 3,
    )(efeat, kv, vv, C1,
      W_qe, r2(b_qe), W_l1, r2(b_l1), W_l2, r2(b_l2), r2(ln1_g), r2(ln1_b),
      W_ke, r2(b_ke), W_ve, r2(b_ve))

    out = pl.pallas_call(
        functools.partial(_node_body, inv_sqrt_qd),
        grid=(grid_n,),
        in_specs=[row_spec, row_spec, _full((M, D)), _full((M, D)),
                  pl.BlockSpec((NB, M), lambda i: (i, 0)),
                  _full((D, D)), _full((1, D)), _full((D, D)), _full((1, D)),
                  _full((1, D)), _full((1, D)), _full((D, H)), _full((1, H))],
        out_specs=pl.BlockSpec((NB, H), lambda i: (i, 0)),
        out_shape=jax.ShapeDtypeStruct((N, H), jnp.float32),
    )(feat_v, q2, k2, v2, C2,
      W_l3, r2(b_l3), W_l4, r2(b_l4), r2(ln2_g), r2(ln2_b), W_mlp, r2(b_mlp))

    return out


# speedup vs baseline: 1.2404x; 1.0889x over previous
import functools
import math

import jax
import jax.numpy as jnp
from jax import lax
from jax.experimental import pallas as pl
from jax.experimental.pallas import tpu as pltpu
from jax.experimental.pallas import tpu_sc as plsc


def _build_c_matrices(in_src, in_dst, bias_in, con_src, con_dst, bias_con, M, N):
    info = plsc.get_sparse_core_info()
    NC, NS = info.num_cores, info.num_subcores
    F1 = M * N
    F2 = N * M
    CH = 1_024_000
    PT = CH // NS
    ZB = 16_000
    ZREP = PT // ZB
    NCH1 = -(-F1 // CH)
    NCH2 = -(-F2 // CH)
    exact1 = F1 % ZB == 0
    exact2 = F2 % ZB == 0
    O1 = F1 if exact1 else NCH1 * CH
    O2 = F2 if exact2 else NCH2 * CH

    E = in_src.shape[0]
    lanes = 16
    epad = (-E) % (NS * lanes)
    if epad:
        pad_i = jnp.zeros((epad,), in_src.dtype)
        pad_f = jnp.full((epad,), -1e30, jnp.float32)
        in_src = jnp.concatenate([in_src, pad_i])
        in_dst = jnp.concatenate([in_dst, pad_i])
        con_src = jnp.concatenate([con_src, pad_i])
        con_dst = jnp.concatenate([con_dst, pad_i])
        bias_in = jnp.concatenate([bias_in, pad_f])
        bias_con = jnp.concatenate([bias_con, pad_f])
    EP = (E + epad) // NS
    LOOPS = EP // lanes
    IDXP = ((EP + 127) // 128) * 128

    def body(insrc_h, indst_h, bin_h, consrc_h, condst_h, bcon_h,
             c1_h, c2_h, flat, idx, val, zeros, stage, shared):
        c = lax.axis_index("c")
        s = lax.axis_index("s")
        z16 = jnp.zeros((16,), jnp.float32)
        dvec = CH + 16 * s + lax.iota(jnp.int32, 16)

        def zfill(i, carry):
            zeros[pl.ds(16 * i, 16)] = z16
            return carry
        lax.fori_loop(0, ZB // 16, zfill, 0)
        for t in range(IDXP // 16 - LOOPS):
            val[pl.ds(EP + 16 * t, 16)] = z16
            idx[pl.ds(EP + 16 * t, 16)] = dvec

        for mat in range(2):
            src_h = insrc_h if mat == 0 else consrc_h
            dst_h = indst_h if mat == 0 else condst_h
            b_h = bin_h if mat == 0 else bcon_h
            out_h = c1_h if mat == 0 else c2_h
            ncols = N if mat == 0 else M
            nch = NCH1 if mat == 0 else NCH2
            fsz = O1 if mat == 0 else O2
            per_core = -(-nch // NC)

            pltpu.sync_copy(dst_h.at[pl.ds(s * EP, EP)], flat)
            pltpu.sync_copy(src_h.at[pl.ds(s * EP, EP)], idx.at[pl.ds(0, EP)])
            pltpu.sync_copy(b_h.at[pl.ds(s * EP, EP)], val.at[pl.ds(0, EP)])

            def build(i, carry):
                o = 16 * i
                flat[pl.ds(o, 16)] = flat[pl.ds(o, 16)] * ncols + idx[pl.ds(o, 16)]
                val[pl.ds(o, 16)] = jnp.exp(val[pl.ds(o, 16)])
                return carry
            lax.fori_loop(0, LOOPS, build, 0)

            for jj in range(per_core):
                j = jnp.minimum(per_core * c + jj, nch - 1)
                lo = j * CH
                for z in range(ZREP):
                    pltpu.sync_copy(zeros.at[pl.ds(0, ZB)],
                                    shared.at[pl.ds(s * PT + z * ZB, ZB)])
                plsc.subcore_barrier()

                def mkidx(i, carry):
                    o = 16 * i
                    f16 = flat[pl.ds(o, 16)]
                    inr = (f16 >= lo) & (f16 < lo + CH)
                    idx[pl.ds(o, 16)] = jnp.where(inr, f16 - lo, dvec)
                    return carry
                lax.fori_loop(0, LOOPS, mkidx, 0)

                pltpu.sync_copy(val, shared.at[idx], add=True)
                plsc.subcore_barrier()
                for z in range(ZREP):
                    off = s * PT + z * ZB

                    @pl.when(lo + off + ZB <= fsz)
                    def _():
                        pltpu.sync_copy(shared.at[pl.ds(off, ZB)], stage)
                        pltpu.sync_copy(stage, out_h.at[pl.ds(lo + off, ZB)])

    builder = pl.kernel(
        body,
        out_type=[jax.ShapeDtypeStruct((O1,), jnp.float32),
                  jax.ShapeDtypeStruct((O2,), jnp.float32)],
        mesh=plsc.VectorSubcoreMesh(core_axis_name="c", subcore_axis_name="s"),
        scratch_types=[
            pltpu.VMEM((EP,), jnp.int32),
            pltpu.VMEM((IDXP,), jnp.int32),
            pltpu.VMEM((IDXP,), jnp.float32),
            pltpu.VMEM((ZB,), jnp.float32),
            pltpu.VMEM((ZB,), jnp.float32),
            pltpu.VMEM_SHARED((CH + 384,), jnp.float32),
        ],
    )
    c1f, c2f = builder(in_src, in_dst, bias_in, con_src, con_dst, bias_con)
    return c1f[:F1].reshape(M, N), c2f[:F2].reshape(N, M)


def _ln(x, g, b):
    m = jnp.mean(x, axis=-1, keepdims=True)
    v = jnp.mean((x - m) * (x - m), axis=-1, keepdims=True)
    return (x - m) * jax.lax.rsqrt(v + 1e-5) * g + b


def _vtx_body(K, vfeat_ref, cent_ref, uniq_ref, Wv_ref, bv_ref, cs_ref, un_ref,
              Wk_ref, bk_ref, Wvv_ref, bvv_ref, Wq2_ref, bq2_ref,
              feat_ref, k_ref, v_ref, q2_ref):
    x = vfeat_ref[...]
    R = x.shape[0]
    f = jnp.dot(x, Wv_ref[...], preferred_element_type=jnp.float32) + bv_ref[...]
    oh_c = (jax.lax.broadcasted_iota(jnp.int32, (R, K), 1) == cent_ref[...]).astype(jnp.float32)
    f = f + jnp.dot(oh_c, cs_ref[...], preferred_element_type=jnp.float32)
    oh_u = (jax.lax.broadcasted_iota(jnp.int32, (R, K), 1) == uniq_ref[...]).astype(jnp.float32)
    f = f + jnp.dot(oh_u, un_ref[...], preferred_element_type=jnp.float32)
    feat_ref[...] = f
    k_ref[...] = jnp.dot(f, Wk_ref[...], preferred_element_type=jnp.float32) + bk_ref[...]
    v_ref[...] = jnp.dot(f, Wvv_ref[...], preferred_element_type=jnp.float32) + bvv_ref[...]
    q2_ref[...] = jnp.dot(f, Wq2_ref[...], preferred_element_type=jnp.float32) + bq2_ref[...]


def _edge_body(inv_sqrt_qd,
               efeat_ref, k_ref, v_ref, C_ref,
               Wqe_ref, bqe_ref, Wl1_ref, bl1_ref, Wl2_ref, bl2_ref,
               ln1g_ref, ln1b_ref, Wke_ref, bke_ref, Wve_ref, bve_ref,
               feat_e_ref, k2_ref, v2_ref):
    ef = efeat_ref[...]
    q = jnp.dot(ef, Wqe_ref[...], preferred_element_type=jnp.float32) + bqe_ref[...]
    S = jax.lax.dot_general(q, k_ref[...], (((1,), (1,)), ((), ())),
                            preferred_element_type=jnp.float32)
    A = jnp.where(S >= 0, S, 0.01 * S) * inv_sqrt_qd
    m = jnp.max(A, axis=1, keepdims=True)
    P = jnp.exp(A - m) * C_ref[...]
    s = jnp.sum(P, axis=1, keepdims=True)
    h = jnp.dot(P, v_ref[...], preferred_element_type=jnp.float32) / jnp.maximum(s, 1e-30)
    x = _ln(h + ef, ln1g_ref[...], ln1b_ref[...])
    f = jnp.dot(jax.nn.relu(jnp.dot(x, Wl1_ref[...], preferred_element_type=jnp.float32) + bl1_ref[...]),
                Wl2_ref[...], preferred_element_type=jnp.float32) + bl2_ref[...]
    fe = _ln(f + x, ln1g_ref[...], ln1b_ref[...])
    feat_e_ref[...] = fe
    k2_ref[...] = jnp.dot(fe, Wke_ref[...], preferred_element_type=jnp.float32) + bke_ref[...]
    v2_ref[...] = jnp.dot(fe, Wve_ref[...], preferred_element_type=jnp.float32) + bve_ref[...]


def _node_body(inv_sqrt_qd,
               featv_ref, q2_ref, k2_ref, v2_ref, C_ref,
               Wl3_ref, bl3_ref, Wl4_ref, bl4_ref,
               ln2g_ref, ln2b_ref, Wmlp_ref, bmlp_ref,
               out_ref):
    S = jax.lax.dot_general(q2_ref[...], k2_ref[...], (((1,), (1,)), ((), ())),
                            preferred_element_type=jnp.float32)
    A = jnp.where(S >= 0, S, 0.01 * S) * inv_sqrt_qd
    m = jnp.max(A, axis=1, keepdims=True)
    P = jnp.exp(A - m) * C_ref[...]
    s = jnp.sum(P, axis=1, keepdims=True)
    h = jnp.dot(P, v2_ref[...], preferred_element_type=jnp.float32) / jnp.maximum(s, 1e-30)
    y = _ln(h + featv_ref[...], ln2g_ref[...], ln2b_ref[...])
    f2 = jnp.dot(jax.nn.relu(jnp.dot(y, Wl3_ref[...], preferred_element_type=jnp.float32) + bl3_ref[...]),
                 Wl4_ref[...], preferred_element_type=jnp.float32) + bl4_ref[...]
    fv2 = _ln(f2 + y, ln2g_ref[...], ln2b_ref[...])
    out_ref[...] = jnp.dot(fv2, Wmlp_ref[...], preferred_element_type=jnp.float32) + bmlp_ref[...]


def _full(shape):
    return pl.BlockSpec(shape, lambda i: (0,) * len(shape))


def kernel(vfeat, efeat, bias_in, bias_con, W_vtx1, b_vtx1, cs_emb, un_emb,
           W_kv, b_kv, W_vv, b_vv, W_qe, b_qe, W_ke, b_ke, W_ve, b_ve,
           W_qv, b_qv, ln1_g, ln1_b, ln2_g, ln2_b, W_l1, b_l1, W_l2, b_l2,
           W_l3, b_l3, W_l4, b_l4, W_mlp, b_mlp,
           centrality_values, uniqueness, in_src, in_dst, con_src, con_dst):
    N, D = vfeat.shape
    M = efeat.shape[0]
    K = cs_emb.shape[0]
    H = W_mlp.shape[1]
    qd = W_kv.shape[1]
    inv_sqrt_qd = 1.0 / math.sqrt(qd)

    r2 = lambda a: a.reshape(1, -1)

    C1, C2 = _build_c_matrices(in_src, in_dst, bias_in,
                               con_src, con_dst, bias_con, M, N)

    NB = 1000 if N % 1000 == 0 else N
    grid_n = N // NB
    cent2 = centrality_values.reshape(N, 1)
    uniq2 = uniqueness.reshape(N, 1)
    row_spec = pl.BlockSpec((NB, D), lambda i: (i, 0))
    idx_spec = pl.BlockSpec((NB, 1), lambda i: (i, 0))
    feat_v, kv, vv, q2 = pl.pallas_call(
        functools.partial(_vtx_body, K),
        grid=(grid_n,),
        in_specs=[row_spec, idx_spec, idx_spec,
                  _full((D, D)), _full((1, D)), _full((K, D)), _full((K, D)),
                  _full((D, D)), _full((1, D)), _full((D, D)), _full((1, D)),
                  _full((D, D)), _full((1, D))],
        out_specs=[row_spec, row_spec, row_spec, row_spec],
        out_shape=[jax.ShapeDtypeStruct((N, D), jnp.float32)] * 4,
    )(vfeat, cent2, uniq2, W_vtx1, r2(b_vtx1), cs_emb, un_emb,
      W_kv, r2(b_kv), W_vv, r2(b_vv), W_qv, r2(b_qv))

    MB = 200 if M % 200 == 0 else M
    grid_m = M // MB
    mrow_spec = pl.BlockSpec((MB, D), lambda i: (i, 0))
    feat_e, k2, v2 = pl.pallas_call(
        functools.partial(_edge_body, inv_sqrt_qd),
        grid=(grid_m,),
        in_specs=[mrow_spec, _full((N, D)), _full((N, D)),
                  pl.BlockSpec((MB, N), lambda i: (i, 0)),
                  _full((D, D)), _full((1, D)), _full((D, D)), _full((1, D)),
                  _full((D, D)), _full((1, D)), _full((1, D)), _full((1, D)),
                  _full((D, D)), _full((1, D)), _full((D, D)), _full((1, D))],
        out_specs=[mrow_spec, mrow_spec, mrow_spec],
        out_shape=[jax.ShapeDtypeStruct((M, D), jnp.float32)] * 3,
    )(efeat, kv, vv, C1,
      W_qe, r2(b_qe), W_l1, r2(b_l1), W_l2, r2(b_l2), r2(ln1_g), r2(ln1_b),
      W_ke, r2(b_ke), W_ve, r2(b_ve))

    out = pl.pallas_call(
        functools.partial(_node_body, inv_sqrt_qd),
        grid=(grid_n,),
        in_specs=[row_spec, row_spec, _full((M, D)), _full((M, D)),
                  pl.BlockSpec((NB, M), lambda i: (i, 0)),
                  _full((D, D)), _full((1, D)), _full((D, D)), _full((1, D)),
                  _full((1, D)), _full((1, D)), _full((D, H)), _full((1, H))],
        out_specs=pl.BlockSpec((NB, H), lambda i: (i, 0)),
        out_shape=jax.ShapeDtypeStruct((N, H), jnp.float32),
    )(feat_v, q2, k2, v2, C2,
      W_l3, r2(b_l3), W_l4, r2(b_l4), r2(ln2_g), r2(ln2_b), W_mlp, r2(b_mlp))

    return out


# speedup vs baseline: 1.2412x; 1.0006x over previous
import functools
import math

import jax
import jax.numpy as jnp
from jax import lax
from jax.experimental import pallas as pl
from jax.experimental.pallas import tpu as pltpu
from jax.experimental.pallas import tpu_sc as plsc


def _build_c_matrices(in_src, in_dst, bias_in, con_src, con_dst, bias_con, M, N):
    info = plsc.get_sparse_core_info()
    NC, NS = info.num_cores, info.num_subcores
    F1 = M * N
    F2 = N * M
    CH = 1_024_000
    PT = CH // NS
    ZB = 16_000
    ZREP = PT // ZB
    NCH1 = -(-F1 // CH)
    NCH2 = -(-F2 // CH)
    exact1 = F1 % ZB == 0
    exact2 = F2 % ZB == 0
    O1 = F1 if exact1 else NCH1 * CH
    O2 = F2 if exact2 else NCH2 * CH

    E = in_src.shape[0]
    lanes = 16
    epad = (-E) % (NS * lanes)
    if epad:
        pad_i = jnp.zeros((epad,), in_src.dtype)
        pad_f = jnp.full((epad,), -1e30, jnp.float32)
        in_src = jnp.concatenate([in_src, pad_i])
        in_dst = jnp.concatenate([in_dst, pad_i])
        con_src = jnp.concatenate([con_src, pad_i])
        con_dst = jnp.concatenate([con_dst, pad_i])
        bias_in = jnp.concatenate([bias_in, pad_f])
        bias_con = jnp.concatenate([bias_con, pad_f])
    EP = (E + epad) // NS
    LOOPS = EP // lanes
    IDXP = ((EP + 127) // 128) * 128

    def body(insrc_h, indst_h, bin_h, consrc_h, condst_h, bcon_h,
             c1_h, c2_h, flat, idx, val, zeros, stage, shared):
        c = lax.axis_index("c")
        s = lax.axis_index("s")
        z16 = jnp.zeros((16,), jnp.float32)
        dvec = CH + 16 * s + lax.iota(jnp.int32, 16)

        def zfill(i, carry):
            zeros[pl.ds(16 * i, 16)] = z16
            return carry
        lax.fori_loop(0, ZB // 16, zfill, 0)
        for t in range(IDXP // 16 - LOOPS):
            val[pl.ds(EP + 16 * t, 16)] = z16
            idx[pl.ds(EP + 16 * t, 16)] = dvec

        for mat in range(2):
            src_h = insrc_h if mat == 0 else consrc_h
            dst_h = indst_h if mat == 0 else condst_h
            b_h = bin_h if mat == 0 else bcon_h
            out_h = c1_h if mat == 0 else c2_h
            ncols = N if mat == 0 else M
            nch = NCH1 if mat == 0 else NCH2
            fsz = O1 if mat == 0 else O2
            per_core = -(-nch // NC)

            pltpu.sync_copy(dst_h.at[pl.ds(s * EP, EP)], flat)
            pltpu.sync_copy(src_h.at[pl.ds(s * EP, EP)], idx.at[pl.ds(0, EP)])
            pltpu.sync_copy(b_h.at[pl.ds(s * EP, EP)], val.at[pl.ds(0, EP)])

            def build(i, carry):
                o = 16 * i
                flat[pl.ds(o, 16)] = flat[pl.ds(o, 16)] * ncols + idx[pl.ds(o, 16)]
                val[pl.ds(o, 16)] = jnp.exp(val[pl.ds(o, 16)])
                return carry
            lax.fori_loop(0, LOOPS, build, 0)

            for jj in range(per_core):
                j = jnp.minimum(per_core * c + jj, nch - 1)
                lo = j * CH
                for z in range(ZREP):
                    pltpu.sync_copy(zeros.at[pl.ds(0, ZB)],
                                    shared.at[pl.ds(s * PT + z * ZB, ZB)])
                plsc.subcore_barrier()

                def mkidx(i, carry):
                    o = 16 * i
                    f16 = flat[pl.ds(o, 16)]
                    inr = (f16 >= lo) & (f16 < lo + CH)
                    idx[pl.ds(o, 16)] = jnp.where(inr, f16 - lo, dvec)
                    return carry
                lax.fori_loop(0, LOOPS, mkidx, 0)

                pltpu.sync_copy(val, shared.at[idx], add=True)
                plsc.subcore_barrier()
                for z in range(ZREP):
                    off = s * PT + z * ZB

                    @pl.when(lo + off + ZB <= fsz)
                    def _():
                        pltpu.sync_copy(shared.at[pl.ds(off, ZB)], stage)
                        pltpu.sync_copy(stage, out_h.at[pl.ds(lo + off, ZB)])

    builder = pl.kernel(
        body,
        out_type=[jax.ShapeDtypeStruct((O1,), jnp.float32),
                  jax.ShapeDtypeStruct((O2,), jnp.float32)],
        mesh=plsc.VectorSubcoreMesh(core_axis_name="c", subcore_axis_name="s"),
        scratch_types=[
            pltpu.VMEM((EP,), jnp.int32),
            pltpu.VMEM((IDXP,), jnp.int32),
            pltpu.VMEM((IDXP,), jnp.float32),
            pltpu.VMEM((ZB,), jnp.float32),
            pltpu.VMEM((ZB,), jnp.float32),
            pltpu.VMEM_SHARED((CH + 384,), jnp.float32),
        ],
    )
    c1f, c2f = builder(in_src, in_dst, bias_in, con_src, con_dst, bias_con)
    return c1f[:F1].reshape(M, N), c2f[:F2].reshape(N, M)


def _ln(x, g, b):
    m = jnp.mean(x, axis=-1, keepdims=True)
    v = jnp.mean((x - m) * (x - m), axis=-1, keepdims=True)
    return (x - m) * jax.lax.rsqrt(v + 1e-5) * g + b


def _vtx_body(K, vfeat_ref, cent_ref, uniq_ref, Wv_ref, bv_ref, cs_ref, un_ref,
              Wk_ref, bk_ref, Wvv_ref, bvv_ref, Wq2_ref, bq2_ref,
              feat_ref, k_ref, v_ref, q2_ref):
    x = vfeat_ref[...]
    R = x.shape[0]
    f = jnp.dot(x, Wv_ref[...], preferred_element_type=jnp.float32) + bv_ref[...]
    oh_c = (jax.lax.broadcasted_iota(jnp.int32, (R, K), 1) == cent_ref[...]).astype(jnp.float32)
    f = f + jnp.dot(oh_c, cs_ref[...], preferred_element_type=jnp.float32)
    oh_u = (jax.lax.broadcasted_iota(jnp.int32, (R, K), 1) == uniq_ref[...]).astype(jnp.float32)
    f = f + jnp.dot(oh_u, un_ref[...], preferred_element_type=jnp.float32)
    feat_ref[...] = f
    k_ref[...] = jnp.dot(f, Wk_ref[...], preferred_element_type=jnp.float32) + bk_ref[...]
    v_ref[...] = jnp.dot(f, Wvv_ref[...], preferred_element_type=jnp.float32) + bvv_ref[...]
    q2_ref[...] = jnp.dot(f, Wq2_ref[...], preferred_element_type=jnp.float32) + bq2_ref[...]


def _edge_body(inv_sqrt_qd,
               efeat_ref, k_ref, v_ref, C_ref,
               Wqe_ref, bqe_ref, Wl1_ref, bl1_ref, Wl2_ref, bl2_ref,
               ln1g_ref, ln1b_ref, Wke_ref, bke_ref, Wve_ref, bve_ref,
               feat_e_ref, k2_ref, v2_ref):
    ef = efeat_ref[...]
    q = jnp.dot(ef, Wqe_ref[...], preferred_element_type=jnp.float32) + bqe_ref[...]
    S = jax.lax.dot_general(q, k_ref[...], (((1,), (1,)), ((), ())),
                            preferred_element_type=jnp.float32)
    A = jnp.where(S >= 0, S, 0.01 * S) * inv_sqrt_qd
    m = jnp.max(A, axis=1, keepdims=True)
    P = jnp.exp(A - m) * C_ref[...]
    s = jnp.sum(P, axis=1, keepdims=True)
    h = jnp.dot(P, v_ref[...], preferred_element_type=jnp.float32) / jnp.maximum(s, 1e-30)
    x = _ln(h + ef, ln1g_ref[...], ln1b_ref[...])
    f = jnp.dot(jax.nn.relu(jnp.dot(x, Wl1_ref[...], preferred_element_type=jnp.float32) + bl1_ref[...]),
                Wl2_ref[...], preferred_element_type=jnp.float32) + bl2_ref[...]
    fe = _ln(f + x, ln1g_ref[...], ln1b_ref[...])
    feat_e_ref[...] = fe
    k2_ref[...] = jnp.dot(fe, Wke_ref[...], preferred_element_type=jnp.float32) + bke_ref[...]
    v2_ref[...] = jnp.dot(fe, Wve_ref[...], preferred_element_type=jnp.float32) + bve_ref[...]


def _node_body(inv_sqrt_qd,
               featv_ref, q2_ref, k2_ref, v2_ref, C_ref,
               Wl3_ref, bl3_ref, Wl4_ref, bl4_ref,
               ln2g_ref, ln2b_ref, Wmlp_ref, bmlp_ref,
               out_ref):
    S = jax.lax.dot_general(q2_ref[...], k2_ref[...], (((1,), (1,)), ((), ())),
                            preferred_element_type=jnp.float32)
    A = jnp.where(S >= 0, S, 0.01 * S) * inv_sqrt_qd
    m = jnp.max(A, axis=1, keepdims=True)
    P = jnp.exp(A - m) * C_ref[...]
    s = jnp.sum(P, axis=1, keepdims=True)
    h = jnp.dot(P, v2_ref[...], preferred_element_type=jnp.float32) / jnp.maximum(s, 1e-30)
    y = _ln(h + featv_ref[...], ln2g_ref[...], ln2b_ref[...])
    f2 = jnp.dot(jax.nn.relu(jnp.dot(y, Wl3_ref[...], preferred_element_type=jnp.float32) + bl3_ref[...]),
                 Wl4_ref[...], preferred_element_type=jnp.float32) + bl4_ref[...]
    fv2 = _ln(f2 + y, ln2g_ref[...], ln2b_ref[...])
    out_ref[...] = jnp.dot(fv2, Wmlp_ref[...], preferred_element_type=jnp.float32) + bmlp_ref[...]


def _full(shape):
    return pl.BlockSpec(shape, lambda i: (0,) * len(shape))


def kernel(vfeat, efeat, bias_in, bias_con, W_vtx1, b_vtx1, cs_emb, un_emb,
           W_kv, b_kv, W_vv, b_vv, W_qe, b_qe, W_ke, b_ke, W_ve, b_ve,
           W_qv, b_qv, ln1_g, ln1_b, ln2_g, ln2_b, W_l1, b_l1, W_l2, b_l2,
           W_l3, b_l3, W_l4, b_l4, W_mlp, b_mlp,
           centrality_values, uniqueness, in_src, in_dst, con_src, con_dst):
    N, D = vfeat.shape
    M = efeat.shape[0]
    K = cs_emb.shape[0]
    H = W_mlp.shape[1]
    qd = W_kv.shape[1]
    inv_sqrt_qd = 1.0 / math.sqrt(qd)

    r2 = lambda a: a.reshape(1, -1)

    C1, C2 = _build_c_matrices(in_src, in_dst, bias_in,
                               con_src, con_dst, bias_con, M, N)

    NB = 1000 if N % 1000 == 0 else N
    grid_n = N // NB
    cent2 = centrality_values.reshape(N, 1)
    uniq2 = uniqueness.reshape(N, 1)
    row_spec = pl.BlockSpec((NB, D), lambda i: (i, 0))
    idx_spec = pl.BlockSpec((NB, 1), lambda i: (i, 0))
    feat_v, kv, vv, q2 = pl.pallas_call(
        functools.partial(_vtx_body, K),
        grid=(grid_n,),
        in_specs=[row_spec, idx_spec, idx_spec,
                  _full((D, D)), _full((1, D)), _full((K, D)), _full((K, D)),
                  _full((D, D)), _full((1, D)), _full((D, D)), _full((1, D)),
                  _full((D, D)), _full((1, D))],
        out_specs=[row_spec, row_spec, row_spec, row_spec],
        out_shape=[jax.ShapeDtypeStruct((N, D), jnp.float32)] * 4,
    )(vfeat, cent2, uniq2, W_vtx1, r2(b_vtx1), cs_emb, un_emb,
      W_kv, r2(b_kv), W_vv, r2(b_vv), W_qv, r2(b_qv))

    MB = 200 if M % 200 == 0 else M
    grid_m = M // MB
    mrow_spec = pl.BlockSpec((MB, D), lambda i: (i, 0))
    feat_e, k2, v2 = pl.pallas_call(
        functools.partial(_edge_body, inv_sqrt_qd),
        grid=(grid_m,),
        in_specs=[mrow_spec, _full((N, D)), _full((N, D)),
                  pl.BlockSpec((MB, N), lambda i: (i, 0)),
                  _full((D, D)), _full((1, D)), _full((D, D)), _full((1, D)),
                  _full((D, D)), _full((1, D)), _full((1, D)), _full((1, D)),
                  _full((D, D)), _full((1, D)), _full((D, D)), _full((1, D))],
        out_specs=[mrow_spec, mrow_spec, mrow_spec],
        out_shape=[jax.ShapeDtypeStruct((M, D), jnp.float32)] * 3,
    )(efeat, kv, vv, C1,
      W_qe, r2(b_qe), W_l1, r2(b_l1), W_l2, r2(b_l2), r2(ln1_g), r2(ln1_b),
      W_ke, r2(b_ke), W_ve, r2(b_ve))

    NB3 = NB
    grid_n3 = N // NB3
    row3 = pl.BlockSpec((NB3, D), lambda i: (i, 0))
    out = pl.pallas_call(
        functools.partial(_node_body, inv_sqrt_qd),
        grid=(grid_n3,),
        in_specs=[row3, row3, _full((M, D)), _full((M, D)),
                  pl.BlockSpec((NB3, M), lambda i: (i, 0)),
                  _full((D, D)), _full((1, D)), _full((D, D)), _full((1, D)),
                  _full((1, D)), _full((1, D)), _full((D, H)), _full((1, H))],
        out_specs=pl.BlockSpec((NB3, H), lambda i: (i, 0)),
        out_shape=jax.ShapeDtypeStruct((N, H), jnp.float32),
    )(feat_v, q2, k2, v2, C2,
      W_l3, r2(b_l3), W_l4, r2(b_l4), r2(ln2_g), r2(ln2_b), W_mlp, r2(b_mlp))

    return out
